# baseline (speedup 1.0000x reference)
import jax
import jax.numpy as jnp
from jax.experimental import pallas as pl

_BLK = 2048


def _top2_block(scores):
    blk, n_e = scores.shape
    e_iota = jax.lax.broadcasted_iota(jnp.int32, scores.shape, 1)
    max1 = jnp.max(scores, axis=1, keepdims=True)
    idx1 = jnp.min(jnp.where(scores == max1, e_iota, n_e), axis=1, keepdims=True)
    masked = jnp.where(e_iota == idx1, -jnp.inf, scores)
    max2 = jnp.max(masked, axis=1, keepdims=True)
    idx2 = jnp.min(jnp.where(masked == max2, e_iota, n_e), axis=1, keepdims=True)
    inv_denom = 1.0 / jnp.sum(jnp.exp(scores - max1), axis=1, keepdims=True)
    w1 = inv_denom
    w2 = jnp.exp(max2 - max1) * inv_denom
    k_iota = jax.lax.broadcasted_iota(jnp.int32, (blk, 2), 1)
    wout = jnp.where(k_iota == 0, w1, w2)
    iout = jnp.where(k_iota == 0, idx1, idx2)
    return wout, iout


def _router_kernel(xa_ref, xb_ref, w_ref, wout_ref, iout_ref):
    w = w_ref[...]
    dn = (((1,), (1,)), ((), ()))
    for j, xr in enumerate((xa_ref, xb_ref)):
        x = xr[...]
        scores = jax.lax.dot_general(x, w, dn, preferred_element_type=jnp.float32)
        wout, iout = _top2_block(scores)
        wout_ref[j] = wout
        iout_ref[j] = iout


def kernel(x, W):
    x2 = x.reshape(x.shape[0], -1)
    rows, win = x2.shape
    n_e = W.shape[0]
    blk = min(_BLK, rows // 2)
    steps = rows // (2 * blk)
    wout, iout = pl.pallas_call(
        _router_kernel,
        grid=(steps,),
        in_specs=[
            pl.BlockSpec((blk, win), lambda i: (i, 0)),
            pl.BlockSpec((blk, win), lambda i: (i + steps, 0)),
            pl.BlockSpec((n_e, win), lambda i: (0, 0)),
        ],
        out_specs=[
            pl.BlockSpec((2, blk, 2), lambda i: (0, i, 0)),
            pl.BlockSpec((2, blk, 2), lambda i: (0, i, 0)),
        ],
        out_shape=[
            jax.ShapeDtypeStruct((2, rows // 2, 2), jnp.float32),
            jax.ShapeDtypeStruct((2, rows // 2, 2), jnp.int32),
        ],
    )(x2, x2, W)
    return wout.reshape(rows, 2).astype(x.dtype), iout.reshape(rows, 2)


# speedup vs baseline: 1.0524x; 1.0524x over previous
import jax
import jax.numpy as jnp
from jax.experimental import pallas as pl

_BLK = 4096


def _router_kernel(x_ref, w_ref, wout_ref, iout_ref):
    x = x_ref[...]
    w = w_ref[...]
    scores = jax.lax.dot_general(
        x, w, (((1,), (1,)), ((), ())), preferred_element_type=jnp.float32
    )
    blk, n_e = scores.shape
    e_iota = jax.lax.broadcasted_iota(jnp.int32, scores.shape, 1)

    max1 = jnp.max(scores, axis=1, keepdims=True)
    idx1 = jnp.min(jnp.where(scores == max1, e_iota, n_e), axis=1, keepdims=True)
    masked = jnp.where(e_iota == idx1, -jnp.inf, scores)
    max2 = jnp.max(masked, axis=1, keepdims=True)
    idx2 = jnp.min(jnp.where(masked == max2, e_iota, n_e), axis=1, keepdims=True)

    inv_denom = 1.0 / jnp.sum(jnp.exp(scores - max1), axis=1, keepdims=True)
    w1 = inv_denom
    w2 = jnp.exp(max2 - max1) * inv_denom

    k_iota = jax.lax.broadcasted_iota(jnp.int32, (blk, 2), 1)
    wout_ref[...] = jnp.where(k_iota == 0, w1, w2)
    iout_ref[...] = jnp.where(k_iota == 0, idx1, idx2)


def kernel(x, W):
    x2 = x.reshape(x.shape[0], -1)
    rows, win = x2.shape
    n_e = W.shape[0]
    blk = min(_BLK, rows)
    grid = (rows // blk,)
    wout, iout = pl.pallas_call(
        _router_kernel,
        grid=grid,
        in_specs=[
            pl.BlockSpec((blk, win), lambda i: (i, 0)),
            pl.BlockSpec((n_e, win), lambda i: (0, 0)),
        ],
        out_specs=[
            pl.BlockSpec((blk, 2), lambda i: (i, 0)),
            pl.BlockSpec((blk, 2), lambda i: (i, 0)),
        ],
        out_shape=[
            jax.ShapeDtypeStruct((rows, 2), jnp.float32),
            jax.ShapeDtypeStruct((rows, 2), jnp.int32),
        ],
    )(x2, W)
    return wout.astype(x.dtype), iout


# speedup vs baseline: 1.0556x; 1.0030x over previous
import jax
import jax.numpy as jnp
from jax.experimental import pallas as pl

_BLK = 4096


def _router_kernel(x_ref, w_ref, wout_ref, iout_ref):
    x = x_ref[...]
    w = w_ref[...]
    scores = jax.lax.dot_general(
        x, w, (((1,), (1,)), ((), ())), preferred_element_type=jnp.float32
    )
    blk, n_e = scores.shape
    e_iota = jax.lax.broadcasted_iota(jnp.int32, scores.shape, 1)

    max1 = jnp.max(scores, axis=1, keepdims=True)
    idx1 = jnp.min(jnp.where(scores == max1, e_iota, n_e), axis=1, keepdims=True)
    masked = jnp.where(e_iota == idx1, -jnp.inf, scores)
    max2 = jnp.max(masked, axis=1, keepdims=True)
    idx2 = jnp.min(jnp.where(masked == max2, e_iota, n_e), axis=1, keepdims=True)

    exps = jnp.exp(scores - max1)
    ones = jnp.ones((1, n_e), jnp.float32)
    denom = jax.lax.dot_general(
        exps, ones, (((1,), (1,)), ((), ())), preferred_element_type=jnp.float32
    )
    inv_denom = 1.0 / denom
    w1 = inv_denom
    w2 = jnp.exp(max2 - max1) * inv_denom

    k_iota = jax.lax.broadcasted_iota(jnp.int32, (blk, 2), 1)
    wout_ref[...] = jnp.where(k_iota == 0, w1, w2)
    iout_ref[...] = jnp.where(k_iota == 0, idx1, idx2)


def kernel(x, W):
    x2 = x.reshape(x.shape[0], -1)
    rows, win = x2.shape
    n_e = W.shape[0]
    blk = min(_BLK, rows)
    grid = (rows // blk,)
    wout, iout = pl.pallas_call(
        _router_kernel,
        grid=grid,
        in_specs=[
            pl.BlockSpec((blk, win), lambda i: (i, 0)),
            pl.BlockSpec((n_e, win), lambda i: (0, 0)),
        ],
        out_specs=[
            pl.BlockSpec((blk, 2), lambda i: (i, 0)),
            pl.BlockSpec((blk, 2), lambda i: (i, 0)),
        ],
        out_shape=[
            jax.ShapeDtypeStruct((rows, 2), jnp.float32),
            jax.ShapeDtypeStruct((rows, 2), jnp.int32),
        ],
    )(x2, W)
    return wout.astype(x.dtype), iout
